# baseline (speedup 1.0000x reference)
import jax
import jax.numpy as jnp
from jax.experimental import pallas as pl
from jax.experimental.pallas import tpu as pltpu

RB = 512


def _gcn_block(xt_ref, w_ref, b_ref, a_ref, ot_ref, a_out_ref, xw_ref):
    b = pl.program_id(0)
    j = pl.program_id(1)

    @pl.when(j == 0)
    def _():
        xt = xt_ref[:, b, :]
        xw_ref[...] = jax.lax.dot_general(
            xt, w_ref[...], (((0,), (0,)), ((), ())),
            preferred_element_type=jnp.float32,
        )

    a_blk = a_ref[0]
    acc = jnp.dot(a_blk, xw_ref[...], preferred_element_type=jnp.float32)
    res = jnp.maximum(acc + b_ref[...], 0.0)
    ot_ref[:, b, pl.ds(j * RB, RB)] = res.T
    a_out_ref[0] = a_blk


def kernel(X, A, weight, bias):
    B, N, D = X.shape
    bias2d = bias.reshape(1, D)
    Xt = jnp.transpose(X, (2, 0, 1))
    grid = (B, N // RB)
    out_t, a_out = pl.pallas_call(
        _gcn_block,
        grid=grid,
        in_specs=[
            pl.BlockSpec((D, B, N), lambda b, j: (0, 0, 0)),
            pl.BlockSpec((D, D), lambda b, j: (0, 0)),
            pl.BlockSpec((1, D), lambda b, j: (0, 0)),
            pl.BlockSpec((1, RB, N), lambda b, j: (b, j, 0)),
        ],
        out_specs=[
            pl.BlockSpec((D, B, N), lambda b, j: (0, 0, 0)),
            pl.BlockSpec((1, RB, N), lambda b, j: (b, j, 0)),
        ],
        out_shape=[
            jax.ShapeDtypeStruct((D, B, N), jnp.float32),
            jax.ShapeDtypeStruct((B, N, N), jnp.float32),
        ],
        scratch_shapes=[pltpu.VMEM((N, D), jnp.float32)],
        compiler_params=pltpu.CompilerParams(
            dimension_semantics=("arbitrary", "arbitrary"),
        ),
    )(Xt, weight, bias2d, A)
    out = jnp.transpose(out_t, (1, 2, 0))
    return (out, a_out)


# speedup vs baseline: 1.0662x; 1.0662x over previous
import jax
import jax.numpy as jnp
from jax.experimental import pallas as pl
from jax.experimental.pallas import tpu as pltpu

RB = 1024


def _gcn_block(xt_ref, w_ref, b_ref, a_ref, ot_ref, a_out_ref, xw_ref):
    b = pl.program_id(0)
    j = pl.program_id(1)

    @pl.when(j == 0)
    def _():
        xt = xt_ref[:, b, :]
        xw_ref[...] = jax.lax.dot_general(
            xt, w_ref[...], (((0,), (0,)), ((), ())),
            preferred_element_type=jnp.float32,
        )

    a_blk = a_ref[0]
    acc = jnp.dot(a_blk, xw_ref[...], preferred_element_type=jnp.float32)
    res = jnp.maximum(acc + b_ref[...], 0.0)
    ot_ref[:, b, pl.ds(j * RB, RB)] = res.T
    a_out_ref[0] = a_blk


def kernel(X, A, weight, bias):
    B, N, D = X.shape
    bias2d = bias.reshape(1, D)
    Xt = jnp.transpose(X, (2, 0, 1))
    grid = (B, N // RB)
    out_t, a_out = pl.pallas_call(
        _gcn_block,
        grid=grid,
        in_specs=[
            pl.BlockSpec((D, B, N), lambda b, j: (0, 0, 0)),
            pl.BlockSpec((D, D), lambda b, j: (0, 0)),
            pl.BlockSpec((1, D), lambda b, j: (0, 0)),
            pl.BlockSpec((1, RB, N), lambda b, j: (b, j, 0)),
        ],
        out_specs=[
            pl.BlockSpec((D, B, N), lambda b, j: (0, 0, 0)),
            pl.BlockSpec((1, RB, N), lambda b, j: (b, j, 0)),
        ],
        out_shape=[
            jax.ShapeDtypeStruct((D, B, N), jnp.float32),
            jax.ShapeDtypeStruct((B, N, N), jnp.float32),
        ],
        scratch_shapes=[pltpu.VMEM((N, D), jnp.float32)],
        compiler_params=pltpu.CompilerParams(
            dimension_semantics=("arbitrary", "arbitrary"),
        ),
    )(Xt, weight, bias2d, A)
    out = jnp.transpose(out_t, (1, 2, 0))
    return (out, a_out)
